# baseline (speedup 1.0000x reference)
import functools

import jax
import jax.numpy as jnp
from jax import lax
from jax.experimental import pallas as pl
from jax.experimental.pallas import tpu as pltpu
from jax.experimental.pallas import tpu_sc as plsc

NROWS = 16384
NF = 26
CARD = 100
D = NF * CARD
NC, NS, L = 2, 16, 16
NW = NC * NS
ROWS_PER_W = NROWS // NW
PIECE_R = 128
PIECE_C = 2 * CARD
NBANDS = NF // 2
RVEC = PIECE_R // L
NBUF = 4


def _sc_body(xt_hbm, out_hbm, xall, buf0, buf1, buf2, buf3,
             sem0, sem1, sem2, sem3):
    wid = lax.axis_index("s") * NC + lax.axis_index("c")
    row_base = wid * ROWS_PER_W

    ones = jnp.ones((L,), jnp.float32)
    zeros = jnp.zeros((L,), jnp.float32)
    iota = lax.iota(jnp.int32, L)

    bufs = (buf0, buf1, buf2, buf3)
    sems = (sem0, sem1, sem2, sem3)

    xcp = pltpu.async_copy(
        xt_hbm.at[:, pl.ds(row_base, ROWS_PER_W)], xall, sem0)

    def zero(buf):
        def zbody(r, c):
            for i in range(PIECE_R // L):
                buf[r, pl.ds(i * L, L)] = zeros
            return c

        lax.fori_loop(0, PIECE_C, zbody, 0)

    zero(bufs[0])
    xcp.wait()

    NRP = ROWS_PER_W // PIECE_R
    NP = NBANDS * NRP

    def scat(buf, m, val):
        band = m // NRP
        rpart = m % NRP
        for dd in range(2):
            f = band * 2 + dd
            for i in range(RVEC):
                codes = xall[f, pl.ds(rpart * PIECE_R + i * L, L)]
                plsc.store_scatter(
                    buf, [codes + dd * CARD, iota + i * L], val)

    def piece_dma(buf, m, sem):
        band = m // NRP
        rpart = m % NRP
        pltpu.async_copy(
            buf,
            out_hbm.at[pl.ds(band * PIECE_C, PIECE_C),
                       pl.ds(row_base + rpart * PIECE_R, PIECE_R)],
            sem)

    scat(bufs[0], 0, ones)
    piece_dma(bufs[0], 0, sems[0])
    for b in range(1, NBUF):
        zero(bufs[b])
        scat(bufs[b], b, ones)
        piece_dma(bufs[b], b, sems[b])

    def super_body(k, c):
        for b in range(NBUF):
            m = k * NBUF + b
            buf, sem = bufs[b], sems[b]
            pltpu.make_async_copy(
                buf, out_hbm.at[pl.ds(0, PIECE_C), pl.ds(0, PIECE_R)],
                sem).wait()
            scat(buf, m - NBUF, zeros)
            scat(buf, m, ones)
            piece_dma(buf, m, sem)
        return c

    lax.fori_loop(1, NP // NBUF, super_body, 0)
    for b in range(NBUF):
        pltpu.make_async_copy(
            bufs[b], out_hbm.at[pl.ds(0, PIECE_C), pl.ds(0, PIECE_R)],
            sems[b]).wait()


@functools.partial(jax.jit, donate_argnums=())
def _onehot_t(xt):
    mesh = plsc.VectorSubcoreMesh(
        core_axis_name="c", subcore_axis_name="s", num_cores=NC,
        num_subcores=NS)
    f = pl.kernel(
        _sc_body,
        out_type=jax.ShapeDtypeStruct((D, NROWS), jnp.float32),
        mesh=mesh,
        scratch_types=[
            pltpu.VMEM((NF, ROWS_PER_W), jnp.int32),
            pltpu.VMEM((PIECE_C, PIECE_R), jnp.float32),
            pltpu.VMEM((PIECE_C, PIECE_R), jnp.float32),
            pltpu.VMEM((PIECE_C, PIECE_R), jnp.float32),
            pltpu.VMEM((PIECE_C, PIECE_R), jnp.float32),
            pltpu.SemaphoreType.DMA,
            pltpu.SemaphoreType.DMA,
            pltpu.SemaphoreType.DMA,
            pltpu.SemaphoreType.DMA,
        ],
        compiler_params=pltpu.CompilerParams(needs_layout_passes=False),
    )
    return f(xt)


def kernel(x):
    return _onehot_t(x.T).T


# speedup vs baseline: 1.0126x; 1.0126x over previous
import functools

import jax
import jax.numpy as jnp
from jax import lax
from jax.experimental import pallas as pl
from jax.experimental.pallas import tpu as pltpu
from jax.experimental.pallas import tpu_sc as plsc

NROWS = 16384
NF = 26
CARD = 100
D = NF * CARD
NC, NS, L = 2, 16, 16
NW = NC * NS
ROWS_PER_W = NROWS // NW
PIECE_R = 128
PIECE_C = 2 * CARD
NBANDS = NF // 2
RVEC = PIECE_R // L
NBUF = 2


def _sc_body(xt_hbm, out_hbm, xall, buf0, buf1, sem0, sem1):
    wid = lax.axis_index("s") * NC + lax.axis_index("c")
    row_base = wid * ROWS_PER_W

    ones = jnp.ones((L,), jnp.float32)
    zeros = jnp.zeros((L,), jnp.float32)
    iota = lax.iota(jnp.int32, L)

    bufs = (buf0, buf1)
    sems = (sem0, sem1)

    xcp = pltpu.async_copy(
        xt_hbm.at[:, pl.ds(row_base, ROWS_PER_W)], xall, sem0)

    def zero(buf):
        def zbody(r, c):
            for i in range(PIECE_R // L):
                buf[r, pl.ds(i * L, L)] = zeros
            return c

        lax.fori_loop(0, PIECE_C, zbody, 0)

    zero(bufs[0])
    xcp.wait()

    NRP = ROWS_PER_W // PIECE_R
    NP = NBANDS * NRP

    def scat(buf, m, val):
        band = m // NRP
        rpart = m % NRP
        for dd in range(2):
            f = band * 2 + dd
            for i in range(RVEC):
                codes = xall[f, pl.ds(rpart * PIECE_R + i * L, L)]
                plsc.store_scatter(
                    buf, [codes + dd * CARD, iota + i * L], val)

    def piece_dma(buf, m, sem):
        band = m // NRP
        rpart = m % NRP
        pltpu.async_copy(
            buf,
            out_hbm.at[pl.ds(band * PIECE_C, PIECE_C),
                       pl.ds(row_base + rpart * PIECE_R, PIECE_R)],
            sem)

    scat(bufs[0], 0, ones)
    piece_dma(bufs[0], 0, sems[0])
    for b in range(1, NBUF):
        zero(bufs[b])
        scat(bufs[b], b, ones)
        piece_dma(bufs[b], b, sems[b])

    def super_body(k, c):
        for b in range(NBUF):
            m = k * NBUF + b
            buf, sem = bufs[b], sems[b]
            pltpu.make_async_copy(
                buf, out_hbm.at[pl.ds(0, PIECE_C), pl.ds(0, PIECE_R)],
                sem).wait()
            scat(buf, m - NBUF, zeros)
            scat(buf, m, ones)
            piece_dma(buf, m, sem)
        return c

    lax.fori_loop(1, NP // NBUF, super_body, 0)
    for b in range(NBUF):
        pltpu.make_async_copy(
            bufs[b], out_hbm.at[pl.ds(0, PIECE_C), pl.ds(0, PIECE_R)],
            sems[b]).wait()


@functools.partial(jax.jit, donate_argnums=())
def _onehot_t(xt):
    mesh = plsc.VectorSubcoreMesh(
        core_axis_name="c", subcore_axis_name="s", num_cores=NC,
        num_subcores=NS)
    f = pl.kernel(
        _sc_body,
        out_type=jax.ShapeDtypeStruct((D, NROWS), jnp.float32),
        mesh=mesh,
        scratch_types=[
            pltpu.VMEM((NF, ROWS_PER_W), jnp.int32),
            pltpu.VMEM((PIECE_C, PIECE_R), jnp.float32),
            pltpu.VMEM((PIECE_C, PIECE_R), jnp.float32),
            pltpu.SemaphoreType.DMA,
            pltpu.SemaphoreType.DMA,
        ],
        compiler_params=pltpu.CompilerParams(needs_layout_passes=False),
    )
    return f(xt)


def kernel(x):
    return _onehot_t(x.T).T


# speedup vs baseline: 1.0209x; 1.0082x over previous
import functools

import jax
import jax.numpy as jnp
from jax import lax
from jax.experimental import pallas as pl
from jax.experimental.pallas import tpu as pltpu
from jax.experimental.pallas import tpu_sc as plsc

NROWS = 16384
NF = 26
CARD = 100
D = NF * CARD
NC, NS, L = 2, 16, 16
NW = NC * NS
ROWS_PER_W = NROWS // NW
PIECE_R = 128
PIECE_C = 2 * CARD
NBANDS = NF // 2
RVEC = PIECE_R // L
NBUF = 2


def _sc_body(xt_hbm, out_hbm, xall, buf0, buf1, sem0, sem1):
    wid = lax.axis_index("s") * NC + lax.axis_index("c")
    row_base = wid * ROWS_PER_W

    ones = jnp.ones((L,), jnp.float32)
    zeros = jnp.zeros((L,), jnp.float32)
    iota = lax.iota(jnp.int32, L)

    bufs = (buf0, buf1)
    sems = (sem0, sem1)

    xcp = pltpu.async_copy(
        xt_hbm.at[:, pl.ds(row_base, ROWS_PER_W)], xall, sem0)

    def zero(buf):
        def zbody(r, c):
            for i in range(PIECE_R // L):
                buf[r, pl.ds(i * L, L)] = zeros
            return c

        lax.fori_loop(0, PIECE_C, zbody, 0)

    zero(bufs[0])
    xcp.wait()

    NRP = ROWS_PER_W // PIECE_R
    NP = NBANDS * NRP

    def scat(buf, m, val):
        band = m // NRP
        rpart = m % NRP

        def sbody(i, c):
            for dd in range(2):
                f = band * 2 + dd
                codes = xall[f, pl.ds(rpart * PIECE_R + i * L, L)]
                plsc.store_scatter(
                    buf, [codes + dd * CARD, iota + i * L], val)
            return c

        lax.fori_loop(0, RVEC, sbody, 0)

    def piece_dma(buf, m, sem):
        band = m // NRP
        rpart = m % NRP
        pltpu.async_copy(
            buf,
            out_hbm.at[pl.ds(band * PIECE_C, PIECE_C),
                       pl.ds(row_base + rpart * PIECE_R, PIECE_R)],
            sem)

    scat(bufs[0], 0, ones)
    piece_dma(bufs[0], 0, sems[0])
    for b in range(1, NBUF):
        zero(bufs[b])
        scat(bufs[b], b, ones)
        piece_dma(bufs[b], b, sems[b])

    def super_body(k, c):
        for b in range(NBUF):
            m = k * NBUF + b
            buf, sem = bufs[b], sems[b]
            pltpu.make_async_copy(
                buf, out_hbm.at[pl.ds(0, PIECE_C), pl.ds(0, PIECE_R)],
                sem).wait()
            scat(buf, m - NBUF, zeros)
            scat(buf, m, ones)
            piece_dma(buf, m, sem)
        return c

    lax.fori_loop(1, NP // NBUF, super_body, 0)
    for b in range(NBUF):
        pltpu.make_async_copy(
            bufs[b], out_hbm.at[pl.ds(0, PIECE_C), pl.ds(0, PIECE_R)],
            sems[b]).wait()


@functools.partial(jax.jit, donate_argnums=())
def _onehot_t(xt):
    mesh = plsc.VectorSubcoreMesh(
        core_axis_name="c", subcore_axis_name="s", num_cores=NC,
        num_subcores=NS)
    f = pl.kernel(
        _sc_body,
        out_type=jax.ShapeDtypeStruct((D, NROWS), jnp.float32),
        mesh=mesh,
        scratch_types=[
            pltpu.VMEM((NF, ROWS_PER_W), jnp.int32),
            pltpu.VMEM((PIECE_C, PIECE_R), jnp.float32),
            pltpu.VMEM((PIECE_C, PIECE_R), jnp.float32),
            pltpu.SemaphoreType.DMA,
            pltpu.SemaphoreType.DMA,
        ],
        compiler_params=pltpu.CompilerParams(needs_layout_passes=False),
    )
    return f(xt)


def kernel(x):
    return _onehot_t(x.T).T
